# baseline (speedup 1.0000x reference)
import functools

import jax
import jax.numpy as jnp
from jax import lax
from jax.experimental import pallas as pl
from jax.experimental.pallas import tpu as pltpu
from jax.experimental.pallas import tpu_sc as plsc

N_EXP = 8
BLOCK = 4096
CHUNKS = 4
N_WORKERS = 32
LANES = 16


def _scores_kernel(x_ref, w_ref, b_ref, s_ref):
    x = x_ref[...]
    w = w_ref[...]
    s = jax.lax.dot_general(
        w, x, (((1,), (1,)), ((), ())),
        preferred_element_type=jnp.float32,
    )
    s_ref[...] = s + b_ref[...][:, None]


def _routing_kernel(tok_per_w, s_hbm, base_hbm, out_hbm, s_v, base_v, out_v):
    wid = lax.axis_index("s") * 2 + lax.axis_index("c")
    base = wid * tok_per_w
    pltpu.sync_copy(s_hbm.at[:, pl.ds(base, tok_per_w)], s_v)
    pltpu.sync_copy(base_hbm, base_v)
    chunk_base = jnp.max(base_v[...])

    lane = lax.iota(jnp.int32, LANES)
    zeros = jnp.zeros((LANES,), jnp.float32)

    def body(t, carry):
        svec = [s_v[e, pl.ds(t * LANES, LANES)] for e in range(N_EXP)]
        m = svec[0]
        for e in range(1, N_EXP):
            m = jnp.maximum(m, svec[e])
        denom = jnp.exp(svec[0] - m)
        amax = jnp.where(svec[0] == m, 0, N_EXP)
        for e in range(1, N_EXP):
            denom = denom + jnp.exp(svec[e] - m)
            amax = jnp.minimum(amax, jnp.where(svec[e] == m, e, N_EXP))
        val = 1.0 / denom
        t_vec = t * LANES + lane
        for j in range(N_EXP):
            plsc.store_scatter(
                out_v, [t_vec, jnp.full((LANES,), j, jnp.int32)], zeros)
        plsc.store_scatter(out_v, [t_vec, amax], val)
        return carry

    lax.fori_loop(0, tok_per_w // LANES, body, 0)
    out_base = pl.multiple_of(chunk_base + base, tok_per_w)
    pltpu.sync_copy(out_v, out_hbm.at[pl.ds(out_base, tok_per_w), :])


def kernel(x, W, b):
    n_tokens, c_in = x.shape
    chunk = n_tokens // CHUNKS
    tok_per_w = chunk // N_WORKERS

    out_ref = jax.new_ref(jax.lax.empty((n_tokens, N_EXP), jnp.float32))
    mesh = plsc.VectorSubcoreMesh(core_axis_name="c", subcore_axis_name="s")

    for c in range(CHUNKS):
        scores_c = pl.pallas_call(
            _scores_kernel,
            grid=(chunk // BLOCK,),
            in_specs=[
                pl.BlockSpec((BLOCK, c_in),
                             lambda i, c=c: (c * (chunk // BLOCK) + i, 0)),
                pl.BlockSpec((N_EXP, c_in), lambda i: (0, 0)),
                pl.BlockSpec((N_EXP,), lambda i: (0,)),
            ],
            out_specs=pl.BlockSpec((N_EXP, BLOCK), lambda i: (0, i)),
            out_shape=jax.ShapeDtypeStruct((N_EXP, chunk), jnp.float32),
        )(x, W, b)

        routing = functools.partial(
            pl.kernel,
            mesh=mesh,
            compiler_params=pltpu.CompilerParams(
                needs_layout_passes=False, use_tc_tiling_on_sc=True),
            scratch_types=[
                pltpu.VMEM((N_EXP, tok_per_w), jnp.float32),
                pltpu.VMEM((LANES,), jnp.int32),
                pltpu.VMEM((tok_per_w, N_EXP), jnp.float32),
            ],
        )(functools.partial(_routing_kernel, tok_per_w))
        base_c = jnp.full((LANES,), c * chunk, jnp.int32)
        routing(scores_c, base_c, out_ref)
    return jax.freeze(out_ref)


# speedup vs baseline: 1.3426x; 1.3426x over previous
import functools

import jax
import jax.numpy as jnp
from jax import lax
from jax.experimental import pallas as pl
from jax.experimental.pallas import tpu as pltpu
from jax.experimental.pallas import tpu_sc as plsc

N_EXP = 8
BLOCK = 4096
CHUNKS = 4
N_WORKERS = 32
LANES = 16


def _scores_kernel(x_ref, w_ref, b_ref, s_ref):
    x = x_ref[...]
    w = w_ref[...]
    s = jax.lax.dot_general(
        w, x, (((1,), (1,)), ((), ())),
        preferred_element_type=jnp.float32,
    )
    s_ref[...] = s + b_ref[...][:, None]


def _routing_kernel(tok_per_w, chunk_base, s_hbm, out_hbm, s_v, out_v):
    wid = lax.axis_index("s") * 2 + lax.axis_index("c")
    base = wid * tok_per_w
    pltpu.sync_copy(s_hbm.at[:, pl.ds(base, tok_per_w)], s_v)

    lane = lax.iota(jnp.int32, LANES)
    zeros = jnp.zeros((LANES,), jnp.float32)

    def body(t, carry):
        svec = [s_v[e, pl.ds(t * LANES, LANES)] for e in range(N_EXP)]
        m = svec[0]
        for e in range(1, N_EXP):
            m = jnp.maximum(m, svec[e])
        denom = jnp.exp(svec[0] - m)
        amax = jnp.where(svec[0] == m, 0, N_EXP)
        for e in range(1, N_EXP):
            denom = denom + jnp.exp(svec[e] - m)
            amax = jnp.minimum(amax, jnp.where(svec[e] == m, e, N_EXP))
        val = 1.0 / denom
        t_vec = t * LANES + lane
        for j in range(N_EXP):
            out_v[j, pl.ds(t * LANES, LANES)] = zeros
        plsc.store_scatter(out_v, [amax, t_vec], val)
        return carry

    lax.fori_loop(0, tok_per_w // LANES, body, 0)
    pltpu.sync_copy(
        out_v, out_hbm.at[:, pl.ds(chunk_base + base, tok_per_w)])


def kernel(x, W, b):
    n_tokens, c_in = x.shape
    chunk = n_tokens // CHUNKS
    tok_per_w = chunk // N_WORKERS

    out_ref = jax.new_ref(jax.lax.empty((N_EXP, n_tokens), jnp.float32))
    mesh = plsc.VectorSubcoreMesh(core_axis_name="c", subcore_axis_name="s")

    for c in range(CHUNKS):
        scores_c = pl.pallas_call(
            _scores_kernel,
            grid=(chunk // BLOCK,),
            in_specs=[
                pl.BlockSpec((BLOCK, c_in),
                             lambda i, c=c: (c * (chunk // BLOCK) + i, 0)),
                pl.BlockSpec((N_EXP, c_in), lambda i: (0, 0)),
                pl.BlockSpec((N_EXP,), lambda i: (0,)),
            ],
            out_specs=pl.BlockSpec((N_EXP, BLOCK), lambda i: (0, i)),
            out_shape=jax.ShapeDtypeStruct((N_EXP, chunk), jnp.float32),
        )(x, W, b)

        routing = functools.partial(
            pl.kernel,
            mesh=mesh,
            compiler_params=pltpu.CompilerParams(
                needs_layout_passes=False, use_tc_tiling_on_sc=True),
            scratch_types=[
                pltpu.VMEM((N_EXP, tok_per_w), jnp.float32),
                pltpu.VMEM((N_EXP, tok_per_w), jnp.float32),
            ],
        )(functools.partial(_routing_kernel, tok_per_w, c * chunk))
        routing(scores_c, out_ref)
    return jax.freeze(out_ref).T
